# Initial kernel scaffold; baseline (speedup 1.0000x reference)
import jax
import jax.numpy as jnp
from jax.experimental import pallas as pl


def kernel(input_ids, table):
    raise NotImplementedError("write your pallas kernel here")


# speedup vs baseline: 6.5976x; 6.5976x over previous
import functools

import jax
import jax.numpy as jnp
from jax import lax
from jax.experimental import pallas as pl
from jax.experimental.pallas import tpu as pltpu
from jax.experimental.pallas import tpu_sc as plsc

VOCAB = 1024
D = 128
BATCH = 16384
SEQ = 200
B = BATCH * SEQ

NC = 2
NS = 16
NW = NC * NS
BPW = B // NW

CH = 128
K = 4
CHUNK = CH * K
NCHUNK = BPW // CHUNK

_mesh = plsc.VectorSubcoreMesh(core_axis_name="c", subcore_axis_name="s")


@functools.partial(
    pl.kernel,
    mesh=_mesh,
    out_type=jax.ShapeDtypeStruct((B, D), jnp.float32),
    scratch_types=[
        pltpu.VMEM((K, CH), jnp.int32),
        pltpu.VMEM((CHUNK, D), jnp.float32),
        pltpu.SemaphoreType.DMA,
    ],
)
def _sc_gather(idx_hbm, table_hbm, out_hbm, idx_v, rows_v, sem):
    wid = lax.axis_index("s") * NC + lax.axis_index("c")
    row0 = wid * (BPW // CH)

    def body(i, carry):
        idx_row = row0 + i * K
        out0 = (row0 + i * K) * CH
        pltpu.sync_copy(idx_hbm.at[pl.ds(idx_row, K)], idx_v)
        copies = [
            pltpu.async_copy(
                table_hbm.at[idx_v.at[j]],
                rows_v.at[pl.ds(j * CH, CH)],
                sem,
            )
            for j in range(K)
        ]
        for c in copies:
            c.wait()
        pltpu.sync_copy(rows_v, out_hbm.at[pl.ds(out0, CHUNK)])
        return carry

    lax.fori_loop(0, NCHUNK, body, 0)


def kernel(input_ids, table):
    ids_flat = input_ids.reshape(-1).astype(jnp.int32)
    idx2 = ids_flat.reshape(B // CH, CH)
    out = _sc_gather(idx2, table)
    return out.reshape(BATCH, SEQ, D), input_ids


# speedup vs baseline: 6.6170x; 1.0030x over previous
import functools

import jax
import jax.numpy as jnp
from jax import lax
from jax.experimental import pallas as pl
from jax.experimental.pallas import tpu as pltpu
from jax.experimental.pallas import tpu_sc as plsc

VOCAB = 1024
D = 128
BATCH = 16384
SEQ = 200
B = BATCH * SEQ

NC = 2
NS = 16
NW = NC * NS
BPW = B // NW

CH = 128
K = 2
CHUNK = CH * K
NCHUNK = BPW // CHUNK
NBUF = 2

_mesh = plsc.VectorSubcoreMesh(core_axis_name="c", subcore_axis_name="s")


@functools.partial(
    pl.kernel,
    mesh=_mesh,
    out_type=jax.ShapeDtypeStruct((B, D), jnp.float32),
    scratch_types=[
        pltpu.VMEM((NBUF, K, CH), jnp.int32),
        pltpu.VMEM((NBUF, CHUNK, D), jnp.float32),
        pltpu.SemaphoreType.DMA,
        pltpu.SemaphoreType.DMA,
    ],
)
def _sc_gather(idx_hbm, table_hbm, out_hbm, idx_v, rows_v, sem_g, sem_w):
    wid = lax.axis_index("s") * NC + lax.axis_index("c")
    row0 = wid * (BPW // CH)

    def process(i, b, drain):
        idx_row = row0 + i * K
        pltpu.sync_copy(idx_hbm.at[pl.ds(idx_row, K)], idx_v.at[b])
        if drain:
            pltpu.make_async_copy(
                rows_v.at[b], out_hbm.at[pl.ds(0, CHUNK)], sem_w
            ).wait()
        copies = [
            pltpu.async_copy(
                table_hbm.at[idx_v.at[b, j]],
                rows_v.at[b, pl.ds(j * CH, CH)],
                sem_g,
            )
            for j in range(K)
        ]
        for c in copies:
            c.wait()
        pltpu.async_copy(
            rows_v.at[b], out_hbm.at[pl.ds(idx_row * CH, CHUNK)], sem_w
        )

    for b in range(NBUF):
        process(b, b, drain=False)

    def body(io, carry):
        for b in range(NBUF):
            process(io * NBUF + b, b, drain=True)
        return carry

    lax.fori_loop(1, NCHUNK // NBUF, body, 0)

    for b in range(NBUF):
        pltpu.make_async_copy(
            rows_v.at[b], out_hbm.at[pl.ds(0, CHUNK)], sem_w
        ).wait()


def kernel(input_ids, table):
    ids_flat = input_ids.reshape(-1).astype(jnp.int32)
    idx2 = ids_flat.reshape(B // CH, CH)
    out = _sc_gather(idx2, table)
    return out.reshape(BATCH, SEQ, D), input_ids


# speedup vs baseline: 15.0601x; 2.2760x over previous
import functools

import jax
import jax.numpy as jnp
from jax import lax
from jax.experimental import pallas as pl
from jax.experimental.pallas import tpu as pltpu
from jax.experimental.pallas import tpu_sc as plsc

VOCAB = 1024
D = 128
BATCH = 16384
SEQ = 200
B = BATCH * SEQ

NC = 2
NS = 16
NW = NC * NS
BPW = B // NW

CH = 128
K = 2
CHUNK = CH * K
NCHUNK = BPW // CHUNK
NBUF = 2

_mesh = plsc.VectorSubcoreMesh(core_axis_name="c", subcore_axis_name="s")


@functools.partial(
    pl.kernel,
    mesh=_mesh,
    out_type=jax.ShapeDtypeStruct((B, D), jnp.float32),
    scratch_types=[
        pltpu.VMEM((NBUF, K, CH), jnp.int32),
        pltpu.VMEM((NBUF, CHUNK, D), jnp.float32),
        pltpu.VMEM_SHARED((VOCAB, D), jnp.float32),
        pltpu.SemaphoreType.DMA,
        pltpu.SemaphoreType.DMA,
    ],
)
def _sc_gather(idx_hbm, table_hbm, out_hbm, idx_v, rows_v, tab_sh, sem_g, sem_w):
    sid = lax.axis_index("s")
    wid = sid * NC + lax.axis_index("c")
    row0 = wid * (BPW // CH)

    rpt = VOCAB // NS
    pltpu.sync_copy(
        table_hbm.at[pl.ds(sid * rpt, rpt)], tab_sh.at[pl.ds(sid * rpt, rpt)]
    )
    plsc.subcore_barrier()

    def process(i, b, drain):
        idx_row = row0 + i * K
        pltpu.sync_copy(idx_hbm.at[pl.ds(idx_row, K)], idx_v.at[b])
        if drain:
            pltpu.make_async_copy(
                rows_v.at[b], out_hbm.at[pl.ds(0, CHUNK)], sem_w
            ).wait()
        copies = [
            pltpu.async_copy(
                tab_sh.at[idx_v.at[b, j]],
                rows_v.at[b, pl.ds(j * CH, CH)],
                sem_g,
            )
            for j in range(K)
        ]
        for c in copies:
            c.wait()
        pltpu.async_copy(
            rows_v.at[b], out_hbm.at[pl.ds(idx_row * CH, CHUNK)], sem_w
        )

    for b in range(NBUF):
        process(b, b, drain=False)

    def body(io, carry):
        for b in range(NBUF):
            process(io * NBUF + b, b, drain=True)
        return carry

    lax.fori_loop(1, NCHUNK // NBUF, body, 0)

    for b in range(NBUF):
        pltpu.make_async_copy(
            rows_v.at[b], out_hbm.at[pl.ds(0, CHUNK)], sem_w
        ).wait()


def kernel(input_ids, table):
    ids_flat = input_ids.reshape(-1).astype(jnp.int32)
    idx2 = ids_flat.reshape(B // CH, CH)
    out = _sc_gather(idx2, table)
    return out.reshape(BATCH, SEQ, D), input_ids
